# baseline (speedup 1.0000x reference)
import functools
import types

import jax
import jax.numpy as jnp
from jax import lax
from jax.experimental import pallas as pl
from jax.experimental.pallas import tpu as pltpu
from jax.experimental.pallas import tpu_sc as plsc

N = 10000
E = 320000
D = 128

NC = 2
NS = 16
NW = NC * NS
L = 16

CH = 96
CPW = 106
NCHP = NW * CPW
EP = NCHP * CH
NACC = N + 8
NDOUT = 10112
NSV = 10112

RPT = 624
_WCHUNKS = [(96 * i, 96) for i in range(6)] + [(576, 48)]
_TAIL0 = NS * RPT
_TAILN = N - _TAIL0


def _proj_body(x_ref, w_ref, a1_ref, a2_ref, z_ref, s_ref, t_ref):
    zb = jnp.dot(x_ref[...], w_ref[...], preferred_element_type=jnp.float32)
    z_ref[...] = zb
    s_ref[...] = jnp.dot(zb, a1_ref[...], preferred_element_type=jnp.float32)
    t_ref[...] = jnp.dot(zb, a2_ref[...], preferred_element_type=jnp.float32)


_BN = 400

_proj = pl.pallas_call(
    _proj_body,
    grid=(N // _BN,),
    in_specs=[
        pl.BlockSpec((_BN, D), lambda i: (i, 0)),
        pl.BlockSpec((D, D), lambda i: (0, 0)),
        pl.BlockSpec((D, 1), lambda i: (0, 0)),
        pl.BlockSpec((D, 1), lambda i: (0, 0)),
    ],
    out_specs=[
        pl.BlockSpec((_BN, D), lambda i: (i, 0)),
        pl.BlockSpec((_BN, 1), lambda i: (i, 0)),
        pl.BlockSpec((_BN, 1), lambda i: (i, 0)),
    ],
    out_shape=[
        jax.ShapeDtypeStruct((N, D), jnp.float32),
        jax.ShapeDtypeStruct((N, 1), jnp.float32),
        jax.ShapeDtypeStruct((N, 1), jnp.float32),
    ],
)


def _sc_body(z_hbm, s_hbm, t_hbm, src_hbm, dst_hbm,
             h_out, d_out,
             zbuf, s_v, t_v,
             sidx0, didx0, sidx1, didx1, sidx2, didx2, sidx3, didx3,
             exa, exb, ra, rb,
             h_sh, d_sh,
             is0, is1, is2, is3,
             gsa, gsb, hsa, hsb, dsa, dsb):
    cid = lax.axis_index("c")
    sid = lax.axis_index("s")
    wid = sid * NC + cid

    A = types.SimpleNamespace(exb=exa, rows=ra, gsem=gsa, hsem=hsa, dsem=dsa)
    B = types.SimpleNamespace(exb=exb, rows=rb, gsem=gsb, hsem=hsb, dsem=dsb)
    S0 = types.SimpleNamespace(sidx=sidx0, didx=didx0, isem=is0)
    S1 = types.SimpleNamespace(sidx=sidx1, didx=didx1, isem=is1)
    S2 = types.SimpleNamespace(sidx=sidx2, didx=didx2, isem=is2)
    S3 = types.SimpleNamespace(sidx=sidx3, didx=didx3, isem=is3)

    def _zero_rows(r, carry):
        for j in range(D // L):
            ra[r, pl.ds(j * L, L)] = jnp.zeros((L,), jnp.float32)
        return carry

    lax.fori_loop(0, CH, _zero_rows, 0)

    def _zero_zb(i, carry):
        zbuf[pl.ds(i * L, L)] = jnp.zeros((L,), jnp.float32)
        return carry

    lax.fori_loop(0, 640 // L, _zero_zb, 0)

    for off, sz in _WCHUNKS:
        pltpu.sync_copy(ra.at[pl.ds(0, sz)],
                        h_sh.at[pl.ds(sid * RPT + off, sz)])

    @pl.when(sid < NS - 1)
    def _():
        pltpu.sync_copy(zbuf, d_sh.at[pl.ds(sid * 640, 640)])

    @pl.when(sid == NS - 1)
    def _():
        pltpu.sync_copy(zbuf.at[pl.ds(0, NDOUT - (NS - 1) * 640)],
                        d_sh.at[pl.ds((NS - 1) * 640, NDOUT - (NS - 1) * 640)])

    @pl.when(sid == 0)
    def _():
        pltpu.sync_copy(ra.at[pl.ds(0, NACC - _TAIL0)],
                        h_sh.at[pl.ds(_TAIL0, NACC - _TAIL0)])

    pltpu.sync_copy(s_hbm, s_v.at[pl.ds(0, N)])
    pltpu.sync_copy(t_hbm, t_v.at[pl.ds(0, N)])

    plsc.subcore_barrier()

    def idx_start(k, st):
        base = (wid * CPW + k) * CH
        pltpu.async_copy(src_hbm.at[pl.ds(base, CH)], st.sidx, st.isem)
        pltpu.async_copy(dst_hbm.at[pl.ds(base, CH)], st.didx, st.isem)

    def idx_wait(k, st):
        base = (wid * CPW + k) * CH
        pltpu.make_async_copy(src_hbm.at[pl.ds(base, CH)], st.sidx, st.isem).wait()
        pltpu.make_async_copy(dst_hbm.at[pl.ds(base, CH)], st.didx, st.isem).wait()

    def g_start(buf, st):
        pltpu.async_copy(z_hbm.at[st.sidx], buf.rows, buf.gsem)

    def g_wait(buf, st):
        pltpu.make_async_copy(z_hbm.at[st.sidx], buf.rows, buf.gsem).wait()

    def ex_compute(buf, st):
        for g in range(CH // L):
            sl = pl.ds(g * L, L)
            iv = st.sidx[sl]
            dv = st.didx[sl]
            sv = plsc.load_gather(s_v, [iv])
            tv = plsc.load_gather(t_v, [dv])
            buf.exb[sl] = jnp.exp(sv + tv)

    def scale(buf):
        rbuf, exbuf = buf.rows, buf.exb

        dnums = lax.GatherDimensionNumbers(
            offset_dims=(), collapsed_slice_dims=(0,), start_index_map=(0,))

        def _grp(g, carry):
            exv = exbuf[pl.ds(g * L, L)]
            for u in range(L):
                r = g * L + u
                ex_s = lax.gather(
                    exv, jnp.full((L, 1), u, jnp.int32), dnums, (1,),
                    mode=lax.GatherScatterMode.PROMISE_IN_BOUNDS)
                for j in range(D // L):
                    rbuf[r, pl.ds(j * L, L)] = rbuf[r, pl.ds(j * L, L)] * ex_s
            return carry

        lax.fori_loop(0, CH // L, _grp, 0)

    def sc_start(buf, st):
        pltpu.async_copy(buf.rows, h_sh.at[st.didx], buf.hsem, add=True)
        pltpu.async_copy(buf.exb, d_sh.at[st.didx], buf.dsem, add=True)

    def sc_drain(buf, st):
        pltpu.make_async_copy(buf.rows, h_sh.at[st.didx], buf.hsem).wait()
        pltpu.make_async_copy(buf.exb, d_sh.at[st.didx], buf.dsem).wait()

    def tail_steps(buf, st):
        ex_compute(buf, st)
        g_wait(buf, st)
        scale(buf)
        sc_start(buf, st)

    def step(k, buf, nbuf, s_cur, s_next, s_pre, s_prev):
        sc_drain(nbuf, s_prev)
        idx_start(k + 2, s_pre)
        idx_wait(k + 1, s_next)
        g_start(nbuf, s_next)
        tail_steps(buf, s_cur)

    idx_start(0, S0)
    idx_start(1, S1)
    idx_wait(0, S0)
    g_start(A, S0)
    idx_wait(1, S1)
    g_start(B, S1)
    idx_start(2, S2)
    idx_start(3, S3)
    tail_steps(A, S0)
    sc_drain(A, S0)
    idx_wait(2, S2)
    g_start(A, S2)
    tail_steps(B, S1)

    def _quad(j, carry):
        k0 = 2 + 4 * j
        step(k0 + 0, A, B, S2, S3, S0, S1)
        step(k0 + 1, B, A, S3, S0, S1, S2)
        step(k0 + 2, A, B, S0, S1, S2, S3)
        step(k0 + 3, B, A, S1, S2, S3, S0)
        return carry

    lax.fori_loop(0, (CPW - 6) // 4, _quad, 0)

    step(CPW - 4, A, B, S2, S3, S0, S1)
    step(CPW - 3, B, A, S3, S0, S1, S2)
    sc_drain(B, S3)
    idx_wait(CPW - 1, S1)
    g_start(B, S1)
    tail_steps(A, S0)
    sc_drain(A, S0)
    tail_steps(B, S1)
    sc_drain(B, S1)

    plsc.subcore_barrier()

    for off, sz in _WCHUNKS:
        r0 = sid * RPT + off
        pltpu.sync_copy(h_sh.at[pl.ds(r0, sz)], ra.at[pl.ds(0, sz)])
        pltpu.sync_copy(ra.at[pl.ds(0, sz)], h_out.at[cid, pl.ds(r0, sz)])

    @pl.when(sid == 0)
    def _():
        pltpu.sync_copy(h_sh.at[pl.ds(_TAIL0, _TAILN)], ra.at[pl.ds(0, _TAILN)])
        pltpu.sync_copy(ra.at[pl.ds(0, _TAILN)], h_out.at[cid, pl.ds(_TAIL0, _TAILN)])

    @pl.when(sid == 1)
    def _():
        pltpu.sync_copy(d_sh, s_v)
        pltpu.sync_copy(s_v, d_out.at[cid])


_sc_scatter = functools.partial(
    pl.kernel,
    out_type=[
        jax.ShapeDtypeStruct((NC, N, D), jnp.float32),
        jax.ShapeDtypeStruct((NC, NDOUT), jnp.float32),
    ],
    mesh=plsc.VectorSubcoreMesh(core_axis_name="c", subcore_axis_name="s"),
    compiler_params=pltpu.CompilerParams(needs_layout_passes=False),
    scratch_types=[
        pltpu.VMEM((640,), jnp.float32),
        pltpu.VMEM((NSV,), jnp.float32),
        pltpu.VMEM((NSV,), jnp.float32),
        pltpu.VMEM((CH,), jnp.int32),
        pltpu.VMEM((CH,), jnp.int32),
        pltpu.VMEM((CH,), jnp.int32),
        pltpu.VMEM((CH,), jnp.int32),
        pltpu.VMEM((CH,), jnp.int32),
        pltpu.VMEM((CH,), jnp.int32),
        pltpu.VMEM((CH,), jnp.int32),
        pltpu.VMEM((CH,), jnp.int32),
        pltpu.VMEM((CH,), jnp.float32),
        pltpu.VMEM((CH,), jnp.float32),
        pltpu.VMEM((CH, D), jnp.float32),
        pltpu.VMEM((CH, D), jnp.float32),
        pltpu.VMEM_SHARED((NACC, D), jnp.float32),
        pltpu.VMEM_SHARED((NDOUT,), jnp.float32),
    ] + [pltpu.SemaphoreType.DMA] * 10,
)(_sc_body)


def _comb_body(hp_ref, dp_ref, o_ref):
    num = hp_ref[0] + hp_ref[1]
    den = dp_ref[0] + dp_ref[1]
    o_ref[...] = num / jnp.maximum(den, 1e-16)


_combine = pl.pallas_call(
    _comb_body,
    grid=(N // _BN,),
    in_specs=[
        pl.BlockSpec((NC, _BN, D), lambda i: (0, i, 0)),
        pl.BlockSpec((NC, _BN, 1), lambda i: (0, i, 0)),
    ],
    out_specs=pl.BlockSpec((_BN, D), lambda i: (i, 0)),
    out_shape=jax.ShapeDtypeStruct((N, D), jnp.float32),
)


def kernel(x, edge_index, W, a):
    src = edge_index[0]
    dst = edge_index[1]
    npad = EP - E
    src_p = jnp.concatenate([src, jnp.zeros((npad,), jnp.int32)])
    dst_p = jnp.concatenate([dst, jnp.full((npad,), N, jnp.int32)])
    z, s1, t1 = _proj(x, W, a[:D], a[D:])
    hp, dp = _sc_scatter(z, s1.reshape(N), t1.reshape(N), src_p, dst_p)
    return _combine(hp, dp[:, :N].reshape(NC, N, 1))


# speedup vs baseline: 1.2877x; 1.2877x over previous
import functools
import types

import jax
import jax.numpy as jnp
from jax import lax
from jax.experimental import pallas as pl
from jax.experimental.pallas import tpu as pltpu
from jax.experimental.pallas import tpu_sc as plsc

N = 10000
E = 320000
D = 128

NC = 2
NS = 16
NW = NC * NS
L = 16

CH = 112
CPW = 90
NCHP = NW * CPW
EP = NCHP * CH
NACC = N + 8
NDOUT = 10112
NSV = 10112

RPT = 624
_WCHUNKS = [(104 * i, 104) for i in range(6)]
_TAIL0 = NS * RPT
_TAILN = N - _TAIL0


def _proj_body(x_ref, w_ref, a1_ref, a2_ref, z_ref, s_ref, t_ref):
    zb = jnp.dot(x_ref[...], w_ref[...], preferred_element_type=jnp.float32)
    z_ref[...] = zb
    s_ref[...] = jnp.dot(zb, a1_ref[...], preferred_element_type=jnp.float32)
    t_ref[...] = jnp.dot(zb, a2_ref[...], preferred_element_type=jnp.float32)


_BN = 400

_proj = pl.pallas_call(
    _proj_body,
    grid=(N // _BN,),
    in_specs=[
        pl.BlockSpec((_BN, D), lambda i: (i, 0)),
        pl.BlockSpec((D, D), lambda i: (0, 0)),
        pl.BlockSpec((D, 1), lambda i: (0, 0)),
        pl.BlockSpec((D, 1), lambda i: (0, 0)),
    ],
    out_specs=[
        pl.BlockSpec((_BN, D), lambda i: (i, 0)),
        pl.BlockSpec((_BN, 1), lambda i: (i, 0)),
        pl.BlockSpec((_BN, 1), lambda i: (i, 0)),
    ],
    out_shape=[
        jax.ShapeDtypeStruct((N, D), jnp.float32),
        jax.ShapeDtypeStruct((N, 1), jnp.float32),
        jax.ShapeDtypeStruct((N, 1), jnp.float32),
    ],
)


def _sc_body(z_hbm, s_hbm, t_hbm, src_hbm, dst_hbm,
             h_out, d_out,
             zbuf, s_v, t_v,
             sidx_a, didx_a, sidx_b, didx_b,
             exa, exb, ra, rb,
             h_sh, d_sh,
             isa_, isb_, gsa, gsb, hsa, hsb, dsa, dsb):
    cid = lax.axis_index("c")
    sid = lax.axis_index("s")
    wid = sid * NC + cid

    A = types.SimpleNamespace(sidx=sidx_a, didx=didx_a, exb=exa, rows=ra,
                              isem=isa_, gsem=gsa, hsem=hsa, dsem=dsa)
    B = types.SimpleNamespace(sidx=sidx_b, didx=didx_b, exb=exb, rows=rb,
                              isem=isb_, gsem=gsb, hsem=hsb, dsem=dsb)

    def _zero_rows(r, carry):
        for j in range(D // L):
            ra[r, pl.ds(j * L, L)] = jnp.zeros((L,), jnp.float32)
        return carry

    lax.fori_loop(0, CH, _zero_rows, 0)

    def _zero_zb(i, carry):
        zbuf[pl.ds(i * L, L)] = jnp.zeros((L,), jnp.float32)
        return carry

    lax.fori_loop(0, 640 // L, _zero_zb, 0)

    for off, sz in _WCHUNKS:
        pltpu.sync_copy(ra.at[pl.ds(0, sz)],
                        h_sh.at[pl.ds(sid * RPT + off, sz)])

    @pl.when(sid < NS - 1)
    def _():
        pltpu.sync_copy(zbuf, d_sh.at[pl.ds(sid * 640, 640)])

    @pl.when(sid == NS - 1)
    def _():
        pltpu.sync_copy(zbuf.at[pl.ds(0, NDOUT - (NS - 1) * 640)],
                        d_sh.at[pl.ds((NS - 1) * 640, NDOUT - (NS - 1) * 640)])

    @pl.when(sid == 0)
    def _():
        pltpu.sync_copy(ra.at[pl.ds(0, NACC - _TAIL0)],
                        h_sh.at[pl.ds(_TAIL0, NACC - _TAIL0)])

    pltpu.sync_copy(s_hbm, s_v.at[pl.ds(0, N)])
    pltpu.sync_copy(t_hbm, t_v.at[pl.ds(0, N)])

    plsc.subcore_barrier()

    def idx_start(k, buf):
        base = (wid * CPW + k) * CH
        pltpu.async_copy(src_hbm.at[pl.ds(base, CH)], buf.sidx, buf.isem)
        pltpu.async_copy(dst_hbm.at[pl.ds(base, CH)], buf.didx, buf.isem)

    def idx_wait(k, buf):
        base = (wid * CPW + k) * CH
        pltpu.make_async_copy(src_hbm.at[pl.ds(base, CH)], buf.sidx, buf.isem).wait()
        pltpu.make_async_copy(dst_hbm.at[pl.ds(base, CH)], buf.didx, buf.isem).wait()

    def g_start(buf):
        pltpu.async_copy(z_hbm.at[buf.sidx], buf.rows, buf.gsem)

    def g_wait(buf):
        pltpu.make_async_copy(z_hbm.at[buf.sidx], buf.rows, buf.gsem).wait()

    def ex_compute(buf):
        for g in range(CH // L):
            sl = pl.ds(g * L, L)
            iv = buf.sidx[sl]
            dv = buf.didx[sl]
            sv = plsc.load_gather(s_v, [iv])
            tv = plsc.load_gather(t_v, [dv])
            buf.exb[sl] = jnp.exp(sv + tv)

    def scale(buf):
        rbuf, exbuf = buf.rows, buf.exb

        dnums = lax.GatherDimensionNumbers(
            offset_dims=(), collapsed_slice_dims=(0,), start_index_map=(0,))

        def _grp(g, carry):
            exv = exbuf[pl.ds(g * L, L)]
            for u in range(L):
                r = g * L + u
                ex_s = lax.gather(
                    exv, jnp.full((L, 1), u, jnp.int32), dnums, (1,),
                    mode=lax.GatherScatterMode.PROMISE_IN_BOUNDS)
                for j in range(D // L):
                    rbuf[r, pl.ds(j * L, L)] = rbuf[r, pl.ds(j * L, L)] * ex_s
            return carry

        lax.fori_loop(0, CH // L, _grp, 0)

    def sc_start(buf):
        pltpu.async_copy(buf.rows, h_sh.at[buf.didx], buf.hsem, add=True)
        pltpu.async_copy(buf.exb, d_sh.at[buf.didx], buf.dsem, add=True)

    def sc_drain(buf):
        pltpu.make_async_copy(buf.rows, h_sh.at[buf.didx], buf.hsem).wait()
        pltpu.make_async_copy(buf.exb, d_sh.at[buf.didx], buf.dsem).wait()

    def tail_steps(buf):
        ex_compute(buf)
        g_wait(buf)
        scale(buf)
        sc_start(buf)

    def full_iter(k, buf, nbuf):
        sc_drain(nbuf)
        idx_start(k + 1, nbuf)
        idx_wait(k + 1, nbuf)
        g_start(nbuf)
        tail_steps(buf)

    idx_start(0, A)
    idx_wait(0, A)
    g_start(A)
    idx_start(1, B)
    idx_wait(1, B)
    g_start(B)
    tail_steps(A)

    def _pair(j, carry):
        full_iter(1 + 2 * j, B, A)
        full_iter(2 + 2 * j, A, B)
        return carry

    lax.fori_loop(0, (CPW - 2) // 2, _pair, 0)

    sc_drain(A)
    tail_steps(B)
    sc_drain(B)

    plsc.subcore_barrier()

    for off, sz in _WCHUNKS:
        r0 = sid * RPT + off
        pltpu.sync_copy(h_sh.at[pl.ds(r0, sz)], ra.at[pl.ds(0, sz)])
        pltpu.sync_copy(ra.at[pl.ds(0, sz)], h_out.at[cid, pl.ds(r0, sz)])

    @pl.when(sid == 0)
    def _():
        pltpu.sync_copy(h_sh.at[pl.ds(_TAIL0, _TAILN)], ra.at[pl.ds(0, _TAILN)])
        pltpu.sync_copy(ra.at[pl.ds(0, _TAILN)], h_out.at[cid, pl.ds(_TAIL0, _TAILN)])

    @pl.when(sid == 1)
    def _():
        pltpu.sync_copy(d_sh, s_v)
        pltpu.sync_copy(s_v, d_out.at[cid])


_sc_scatter = functools.partial(
    pl.kernel,
    out_type=[
        jax.ShapeDtypeStruct((NC, N, D), jnp.float32),
        jax.ShapeDtypeStruct((NC, NDOUT), jnp.float32),
    ],
    mesh=plsc.VectorSubcoreMesh(core_axis_name="c", subcore_axis_name="s"),
    compiler_params=pltpu.CompilerParams(needs_layout_passes=False),
    scratch_types=[
        pltpu.VMEM((640,), jnp.float32),
        pltpu.VMEM((NSV,), jnp.float32),
        pltpu.VMEM((NSV,), jnp.float32),
        pltpu.VMEM((CH,), jnp.int32),
        pltpu.VMEM((CH,), jnp.int32),
        pltpu.VMEM((CH,), jnp.int32),
        pltpu.VMEM((CH,), jnp.int32),
        pltpu.VMEM((CH,), jnp.float32),
        pltpu.VMEM((CH,), jnp.float32),
        pltpu.VMEM((CH, D), jnp.float32),
        pltpu.VMEM((CH, D), jnp.float32),
        pltpu.VMEM_SHARED((NACC, D), jnp.float32),
        pltpu.VMEM_SHARED((NDOUT,), jnp.float32),
    ] + [pltpu.SemaphoreType.DMA] * 8,
)(_sc_body)


def _comb_body(hp_ref, dp_ref, o_ref):
    num = hp_ref[0] + hp_ref[1]
    den = dp_ref[0] + dp_ref[1]
    o_ref[...] = num / jnp.maximum(den, 1e-16)


_combine = pl.pallas_call(
    _comb_body,
    grid=(N // _BN,),
    in_specs=[
        pl.BlockSpec((NC, _BN, D), lambda i: (0, i, 0)),
        pl.BlockSpec((NC, _BN, 1), lambda i: (0, i, 0)),
    ],
    out_specs=pl.BlockSpec((_BN, D), lambda i: (i, 0)),
    out_shape=jax.ShapeDtypeStruct((N, D), jnp.float32),
)


def kernel(x, edge_index, W, a):
    src = edge_index[0]
    dst = edge_index[1]
    npad = EP - E
    src_p = jnp.concatenate([src, jnp.zeros((npad,), jnp.int32)])
    dst_p = jnp.concatenate([dst, jnp.full((npad,), N, jnp.int32)])
    z, s1, t1 = _proj(x, W, a[:D], a[D:])
    hp, dp = _sc_scatter(z, s1.reshape(N), t1.reshape(N), src_p, dst_p)
    return _combine(hp, dp[:, :N].reshape(NC, N, 1))
